# baseline (speedup 1.0000x reference)
import functools

import jax
import jax.numpy as jnp
from jax import lax
from jax.experimental import pallas as pl
from jax.experimental.pallas import tpu as pltpu
from jax.experimental.pallas import tpu_sc as plsc

B = 1024
N = 100000
K = 30
KPAD = 32
NC = 2
NS = 16
NW = NC * NS
ROWS_PER_W = B // NW
U = 10
BLK = U * 16
NBLK = N // BLK
NCHUNK = 5
CBLK = NBLK // NCHUNK
CHUNK_E = N // NCHUNK
NBV = NBLK + 5
NMETA = NBV // U
NMV = NMETA + 1
CAND = 512
MERGE_AT = 224
NEG = -1e30


def _merge16(hv, hi, lv, li, cv, ci):
  rv = lax.rev(cv, (0,))
  ri = lax.rev(ci, (0,))
  m = lv >= rv
  nlv = jnp.where(m, lv, rv)
  nli = jnp.where(m, li, ri)
  nlv, nli = plsc.sort_key_val(nlv, nli, descending=True)
  rv = lax.rev(nlv, (0,))
  ri = lax.rev(nli, (0,))
  m = hv >= rv
  uv = jnp.where(m, hv, rv)
  ui = jnp.where(m, hi, ri)
  wv = jnp.where(m, rv, hv)
  wi = jnp.where(m, ri, hi)
  uv, ui = plsc.sort_key_val(uv, ui, descending=True)
  wv, wi = plsc.sort_key_val(wv, wi, descending=True)
  return uv, ui, wv, wi


def _sc_body(targets_hbm, logits_hbm, tvals_hbm, lvals_hbm,
             row_v, bmax_v, mmax_v, cand_v, cand_i, ids_m,
             tv_all, ti_all, lg_all, sems, sem_g):
  wid = lax.axis_index("s") * NC + lax.axis_index("c")
  base = wid * ROWS_PER_W
  lanes = lax.broadcasted_iota(jnp.int32, (16,), 0)
  negv = jnp.full((16,), NEG, jnp.float32)
  iz = jnp.zeros((16,), jnp.int32)

  def fold(cnt, hv, hi, lv, li):
    def mb(k, mc):
      hv, hi, lv, li = mc
      cv = cand_v[pl.ds(k * 16, 16)]
      ci = cand_i[pl.ds(k * 16, 16)]
      valid = (lanes + k * 16) < cnt
      cv = jnp.where(valid, cv, NEG)
      cv, ci = plsc.sort_key_val(cv, ci, descending=True)
      return _merge16(hv, hi, lv, li, cv, ci)
    nb = (cnt + 15) // 16
    return lax.fori_loop(0, nb, mb, (hv, hi, lv, li))

  def scan_vecs(src, nvec):
    def do_merge(thr, cnt, hv, hi, lv, li):
      hv, hi, lv, li = fold(cnt, hv, hi, lv, li)
      return jnp.broadcast_to(lv[15], (16,)), jnp.int32(0), hv, hi, lv, li

    def no_merge(thr, cnt, hv, hi, lv, li):
      return thr, cnt, hv, hi, lv, li

    def body(j, carry):
      thr, cnt, hv, hi, lv, li = carry
      v = src[pl.ds(j * 16, 16)]
      m = v >= thr
      plsc.store_compressed(cand_v.at[pl.ds(cnt, 16)], v, mask=m)
      plsc.store_compressed(cand_i.at[pl.ds(cnt, 16)], j * 16 + lanes, mask=m)
      cnt = cnt + plsc.all_reduce_population_count(m)[0]
      return lax.cond(cnt >= MERGE_AT, do_merge, no_merge,
                      thr, cnt, hv, hi, lv, li)

    carry = lax.fori_loop(0, nvec, body,
                          (negv, jnp.int32(0), negv, iz, negv, iz))
    _, cnt, hv, hi, lv, li = carry
    return fold(cnt, hv, hi, lv, li)

  def set_ids(hq, lq):
    a = lax.shift_right_logical(hq, 4)
    b = lax.shift_right_logical(lq, 4)
    sa, _ = plsc.sort_key_val(a, a, descending=False)
    sb, _ = plsc.sort_key_val(b, b, descending=False)
    rb = lax.rev(sb, (0,))
    idlo = jnp.minimum(sa, rb)
    idhi = jnp.maximum(sa, rb)
    idlo, _ = plsc.sort_key_val(idlo, idlo, descending=False)
    idhi, _ = plsc.sort_key_val(idhi, idhi, descending=False)
    ids_m[pl.ds(0, 16)] = idlo
    ids_m[pl.ds(16, 16)] = idhi
    ids_m[pl.ds(32, 16)] = idhi

  def rescan_fold(src, t16):
    def body(g, carry):
      cnt, prev = carry
      bid = ids_m[pl.ds(g, 16)][0]
      validv = jnp.broadcast_to(bid != prev, (16,))
      cnt = jnp.minimum(cnt, CAND - BLK - 16)
      for u in range(U):
        e = bid * BLK + u * 16
        v = src[pl.ds(e, 16)]
        m = (v >= t16) & validv
        plsc.store_compressed(cand_v.at[pl.ds(cnt, 16)], v, mask=m)
        plsc.store_compressed(cand_i.at[pl.ds(cnt, 16)], e + lanes, mask=m)
        cnt = cnt + plsc.all_reduce_population_count(m)[0]
      return cnt, bid

    cnt, _ = lax.fori_loop(0, KPAD, body, (jnp.int32(0), jnp.int32(-1)))
    return fold(cnt, negv, iz, negv, iz)

  for p in range(NBLK * 16, NBV * 16, 16):
    bmax_v[pl.ds(p, 16)] = negv
  mmax_v[pl.ds(NMETA * 16, 16)] = negv

  def scan_row(row, _):
    roff = pl.multiple_of(row * N, 8)
    cps = [pltpu.async_copy(
        targets_hbm.at[pl.ds(roff + c * CHUNK_E, CHUNK_E)],
        row_v.at[pl.ds(c * CHUNK_E, CHUNK_E)], sems[c])
        for c in range(NCHUNK)]

    for c in range(NCHUNK):
      cps[c].wait()

      def _l0(i, _):
        for s in range(5):
          b = i * 5 + s
          e = b * BLK
          bm = row_v[pl.ds(e, 16)]
          for u in range(1, U):
            bm = jnp.maximum(bm, row_v[pl.ds(e + u * 16, 16)])
          bmax_v[pl.ds(b * 16, 16)] = bm
        return 0
      lax.fori_loop(c * (CBLK // 5), (c + 1) * (CBLK // 5), _l0, 0)

    def _l1(i, _):
      for s in range(3):
        mb = i * 3 + s
        e = mb * BLK
        mm = bmax_v[pl.ds(e, 16)]
        for u in range(1, U):
          mm = jnp.maximum(mm, bmax_v[pl.ds(e + u * 16, 16)])
        mmax_v[pl.ds(mb * 16, 16)] = mm
      return 0
    lax.fori_loop(0, NMETA // 3, _l1, 0)

    mhv, mhq, mlv, mlq = scan_vecs(mmax_v, NMV)
    set_ids(mhq, mlq)
    bhv, bhp, blv, blp = rescan_fold(bmax_v, jnp.broadcast_to(mlv[13], (16,)))
    set_ids(bhp, blp)
    hv2, hi2, lv2, li2 = rescan_fold(row_v, jnp.broadcast_to(blv[13], (16,)))

    r = row - base
    goff = row * N
    tv_all[pl.ds(r * KPAD, 16)] = hv2
    tv_all[pl.ds(r * KPAD + 16, 16)] = lv2
    ti_all[pl.ds(r * KPAD, 16)] = hi2 + goff
    ti_all[pl.ds(r * KPAD + 16, 16)] = li2 + goff
    return 0

  lax.fori_loop(base, base + ROWS_PER_W, scan_row, 0)

  copies = []
  for g in range(ROWS_PER_W * KPAD // 128):
    copies.append(pltpu.async_copy(
        logits_hbm.at[ti_all.at[pl.ds(g * 128, 128)]],
        lg_all.at[pl.ds(g * 128, 128)], sem_g))
  for cp in copies:
    cp.wait()

  out_base = wid * ROWS_PER_W * KPAD
  pltpu.sync_copy(tv_all, tvals_hbm.at[pl.ds(out_base, ROWS_PER_W * KPAD)])
  pltpu.sync_copy(lg_all, lvals_hbm.at[pl.ds(out_base, ROWS_PER_W * KPAD)])


def _topk_gather(logits_flat, targets_flat):
  mesh = plsc.VectorSubcoreMesh(core_axis_name="c", subcore_axis_name="s",
                                num_cores=NC, num_subcores=NS)
  f = pl.kernel(
      _sc_body,
      out_type=[jax.ShapeDtypeStruct((B * KPAD,), jnp.float32),
                jax.ShapeDtypeStruct((B * KPAD,), jnp.float32)],
      mesh=mesh,
      compiler_params=pltpu.CompilerParams(needs_layout_passes=False),
      scratch_types=[
          pltpu.VMEM((N,), jnp.float32),
          pltpu.VMEM((NBV * 16,), jnp.float32),
          pltpu.VMEM((NMV * 16,), jnp.float32),
          pltpu.VMEM((CAND,), jnp.float32),
          pltpu.VMEM((CAND,), jnp.int32),
          pltpu.VMEM((48,), jnp.int32),
          pltpu.VMEM((ROWS_PER_W * KPAD,), jnp.float32),
          pltpu.VMEM((ROWS_PER_W * KPAD,), jnp.int32),
          pltpu.VMEM((ROWS_PER_W * KPAD,), jnp.float32),
          [pltpu.SemaphoreType.DMA] * NCHUNK,
          pltpu.SemaphoreType.DMA,
      ],
  )
  return f(targets_flat, logits_flat)


def _tc_body(t_ref, l_ref, o_ref):
  t = t_ref[...]
  l = l_ref[...]
  mask = lax.broadcasted_iota(jnp.int32, (B, KPAD), 1) < K
  tm = jnp.where(mask, t, NEG)
  lm = jnp.where(mask, l, NEG)
  tmax = jnp.max(tm, axis=1, keepdims=True)
  lmax = jnp.max(lm, axis=1, keepdims=True)
  te = jnp.where(mask, jnp.exp(tm - tmax), 0.0)
  le = jnp.where(mask, jnp.exp(lm - lmax), 0.0)
  ts = jnp.sum(te, axis=1, keepdims=True)
  ls = jnp.sum(le, axis=1, keepdims=True)
  logt = (tm - tmax) - jnp.log(ts)
  logp = (lm - lmax) - jnp.log(ls)
  pw = jnp.where(mask, (te / ts) * (logt - logp), 0.0)
  o_ref[...] = jnp.full((1, 1), jnp.sum(pw) / B, jnp.float32)


@jax.jit
def kernel(logits, targets):
  tv, lv = _topk_gather(logits.reshape(-1), targets.reshape(-1))
  loss = pl.pallas_call(
      _tc_body,
      out_shape=jax.ShapeDtypeStruct((1, 1), jnp.float32),
  )(tv.reshape(B, KPAD), lv.reshape(B, KPAD))
  return loss[0, 0]


# speedup vs baseline: 1.2731x; 1.2731x over previous
import functools

import jax
import jax.numpy as jnp
from jax import lax
from jax.experimental import pallas as pl
from jax.experimental.pallas import tpu as pltpu
from jax.experimental.pallas import tpu_sc as plsc

B = 1024
N = 100000
K = 30
KPAD = 32
NC = 2
NS = 16
NW = NC * NS
ROWS_PER_W = B // NW
U = 10
BLK = U * 16
NBLK = N // BLK
NCHUNK = 5
CBLK = NBLK // NCHUNK
CHUNK_E = N // NCHUNK
NBV = NBLK + 5
NMETA = NBV // U
NMV = NMETA + 1
CAND = 512
MERGE_AT = 224
NEG = -1e30


def _merge16(hv, hi, lv, li, cv, ci):
  rv = lax.rev(cv, (0,))
  ri = lax.rev(ci, (0,))
  m = lv >= rv
  nlv = jnp.where(m, lv, rv)
  nli = jnp.where(m, li, ri)
  nlv, nli = plsc.sort_key_val(nlv, nli, descending=True)
  rv = lax.rev(nlv, (0,))
  ri = lax.rev(nli, (0,))
  m = hv >= rv
  uv = jnp.where(m, hv, rv)
  ui = jnp.where(m, hi, ri)
  wv = jnp.where(m, rv, hv)
  wi = jnp.where(m, ri, hi)
  uv, ui = plsc.sort_key_val(uv, ui, descending=True)
  wv, wi = plsc.sort_key_val(wv, wi, descending=True)
  return uv, ui, wv, wi


def _sc_body(targets_hbm, logits_hbm, tvals_hbm, lvals_hbm,
             row_v, bmax_v, mmax_v, cand_v, cand_i, ids_m,
             tv_all, ti_all, lg_all, sem_g):
  wid = lax.axis_index("s") * NC + lax.axis_index("c")
  base = wid * ROWS_PER_W
  lanes = lax.broadcasted_iota(jnp.int32, (16,), 0)
  negv = jnp.full((16,), NEG, jnp.float32)
  iz = jnp.zeros((16,), jnp.int32)

  def fold(cnt, hv, hi, lv, li):
    def mb(k, mc):
      hv, hi, lv, li = mc
      cv = cand_v[pl.ds(k * 16, 16)]
      ci = cand_i[pl.ds(k * 16, 16)]
      valid = (lanes + k * 16) < cnt
      cv = jnp.where(valid, cv, NEG)
      cv, ci = plsc.sort_key_val(cv, ci, descending=True)
      return _merge16(hv, hi, lv, li, cv, ci)
    nb = (cnt + 15) // 16
    return lax.fori_loop(0, nb, mb, (hv, hi, lv, li))

  def scan_vecs(src, nvec):
    def do_merge(thr, cnt, hv, hi, lv, li):
      hv, hi, lv, li = fold(cnt, hv, hi, lv, li)
      return jnp.broadcast_to(lv[15], (16,)), jnp.int32(0), hv, hi, lv, li

    def no_merge(thr, cnt, hv, hi, lv, li):
      return thr, cnt, hv, hi, lv, li

    def body(j, carry):
      thr, cnt, hv, hi, lv, li = carry
      v = src[pl.ds(j * 16, 16)]
      m = v >= thr
      plsc.store_compressed(cand_v.at[pl.ds(cnt, 16)], v, mask=m)
      plsc.store_compressed(cand_i.at[pl.ds(cnt, 16)], j * 16 + lanes, mask=m)
      cnt = cnt + plsc.all_reduce_population_count(m)[0]
      return lax.cond(cnt >= MERGE_AT, do_merge, no_merge,
                      thr, cnt, hv, hi, lv, li)

    carry = lax.fori_loop(0, nvec, body,
                          (negv, jnp.int32(0), negv, iz, negv, iz))
    _, cnt, hv, hi, lv, li = carry
    return fold(cnt, hv, hi, lv, li)

  def set_ids(hq, lq):
    a = lax.shift_right_logical(hq, 4)
    b = lax.shift_right_logical(lq, 4)
    sa, _ = plsc.sort_key_val(a, a, descending=False)
    sb, _ = plsc.sort_key_val(b, b, descending=False)
    rb = lax.rev(sb, (0,))
    idlo = jnp.minimum(sa, rb)
    idhi = jnp.maximum(sa, rb)
    idlo, _ = plsc.sort_key_val(idlo, idlo, descending=False)
    idhi, _ = plsc.sort_key_val(idhi, idhi, descending=False)
    ids_m[pl.ds(0, 16)] = idlo
    ids_m[pl.ds(16, 16)] = idhi
    ids_m[pl.ds(32, 16)] = idhi

  def rescan_fold(src, t16):
    def body(g, carry):
      cnt, prev = carry
      bid = ids_m[pl.ds(g, 16)][0]
      validv = jnp.broadcast_to(bid != prev, (16,))
      cnt = jnp.minimum(cnt, CAND - BLK - 16)
      for u in range(U):
        e = bid * BLK + u * 16
        v = src[pl.ds(e, 16)]
        m = (v >= t16) & validv
        plsc.store_compressed(cand_v.at[pl.ds(cnt, 16)], v, mask=m)
        plsc.store_compressed(cand_i.at[pl.ds(cnt, 16)], e + lanes, mask=m)
        cnt = cnt + plsc.all_reduce_population_count(m)[0]
      return cnt, bid

    cnt, _ = lax.fori_loop(0, KPAD, body, (jnp.int32(0), jnp.int32(-1)))
    return fold(cnt, negv, iz, negv, iz)

  for p in range(NBLK * 16, NBV * 16, 16):
    bmax_v[pl.ds(p, 16)] = negv
  mmax_v[pl.ds(NMETA * 16, 16)] = negv

  def scan_row(row, _):
    pltpu.sync_copy(targets_hbm.at[row], row_v)

    def _l0(i, _):
      for s in range(5):
        b = i * 5 + s
        e = b * BLK
        bm = row_v[pl.ds(e, 16)]
        for u in range(1, U):
          bm = jnp.maximum(bm, row_v[pl.ds(e + u * 16, 16)])
        bmax_v[pl.ds(b * 16, 16)] = bm
      return 0
    lax.fori_loop(0, NBLK // 5, _l0, 0)

    def _l1(i, _):
      for s in range(3):
        mb = i * 3 + s
        e = mb * BLK
        mm = bmax_v[pl.ds(e, 16)]
        for u in range(1, U):
          mm = jnp.maximum(mm, bmax_v[pl.ds(e + u * 16, 16)])
        mmax_v[pl.ds(mb * 16, 16)] = mm
      return 0
    lax.fori_loop(0, NMETA // 3, _l1, 0)

    mhv, mhq, mlv, mlq = scan_vecs(mmax_v, NMV)
    set_ids(mhq, mlq)
    bhv, bhp, blv, blp = rescan_fold(bmax_v, jnp.broadcast_to(mlv[13], (16,)))
    set_ids(bhp, blp)
    hv2, hi2, lv2, li2 = rescan_fold(row_v, jnp.broadcast_to(blv[13], (16,)))

    r = row - base
    goff = row * N
    tv_all[pl.ds(r * KPAD, 16)] = hv2
    tv_all[pl.ds(r * KPAD + 16, 16)] = lv2
    ti_all[pl.ds(r * KPAD, 16)] = hi2 + goff
    ti_all[pl.ds(r * KPAD + 16, 16)] = li2 + goff
    return 0

  lax.fori_loop(base, base + ROWS_PER_W, scan_row, 0)

  copies = []
  for g in range(ROWS_PER_W * KPAD // 128):
    copies.append(pltpu.async_copy(
        logits_hbm.at[ti_all.at[pl.ds(g * 128, 128)]],
        lg_all.at[pl.ds(g * 128, 128)], sem_g))
  for cp in copies:
    cp.wait()

  out_base = wid * ROWS_PER_W * KPAD
  pltpu.sync_copy(tv_all, tvals_hbm.at[pl.ds(out_base, ROWS_PER_W * KPAD)])
  pltpu.sync_copy(lg_all, lvals_hbm.at[pl.ds(out_base, ROWS_PER_W * KPAD)])


def _topk_gather(logits_flat, targets):
  mesh = plsc.VectorSubcoreMesh(core_axis_name="c", subcore_axis_name="s",
                                num_cores=NC, num_subcores=NS)
  f = pl.kernel(
      _sc_body,
      out_type=[jax.ShapeDtypeStruct((B * KPAD,), jnp.float32),
                jax.ShapeDtypeStruct((B * KPAD,), jnp.float32)],
      mesh=mesh,
      compiler_params=pltpu.CompilerParams(needs_layout_passes=False),
      scratch_types=[
          pltpu.VMEM((N,), jnp.float32),
          pltpu.VMEM((NBV * 16,), jnp.float32),
          pltpu.VMEM((NMV * 16,), jnp.float32),
          pltpu.VMEM((CAND,), jnp.float32),
          pltpu.VMEM((CAND,), jnp.int32),
          pltpu.VMEM((48,), jnp.int32),
          pltpu.VMEM((ROWS_PER_W * KPAD,), jnp.float32),
          pltpu.VMEM((ROWS_PER_W * KPAD,), jnp.int32),
          pltpu.VMEM((ROWS_PER_W * KPAD,), jnp.float32),
          pltpu.SemaphoreType.DMA,
      ],
  )
  return f(targets, logits_flat)


def _tc_body(t_ref, l_ref, o_ref):
  t = t_ref[...]
  l = l_ref[...]
  mask = lax.broadcasted_iota(jnp.int32, (B, KPAD), 1) < K
  tm = jnp.where(mask, t, NEG)
  lm = jnp.where(mask, l, NEG)
  tmax = jnp.max(tm, axis=1, keepdims=True)
  lmax = jnp.max(lm, axis=1, keepdims=True)
  te = jnp.where(mask, jnp.exp(tm - tmax), 0.0)
  le = jnp.where(mask, jnp.exp(lm - lmax), 0.0)
  ts = jnp.sum(te, axis=1, keepdims=True)
  ls = jnp.sum(le, axis=1, keepdims=True)
  logt = (tm - tmax) - jnp.log(ts)
  logp = (lm - lmax) - jnp.log(ls)
  pw = jnp.where(mask, (te / ts) * (logt - logp), 0.0)
  o_ref[...] = jnp.full((1, 1), jnp.sum(pw) / B, jnp.float32)


@jax.jit
def kernel(logits, targets):
  tv, lv = _topk_gather(logits.reshape(-1), targets)
  loss = pl.pallas_call(
      _tc_body,
      out_shape=jax.ShapeDtypeStruct((1, 1), jnp.float32),
  )(tv.reshape(B, KPAD), lv.reshape(B, KPAD))
  return loss[0, 0]
